# Initial kernel scaffold; baseline (speedup 1.0000x reference)
import jax
import jax.numpy as jnp
from jax.experimental import pallas as pl


def kernel(x_author, x_paper, x_term, edge_ap, edge_pa, edge_pt, edge_tp, params):
    raise NotImplementedError("write your pallas kernel here")


# speedup vs baseline: 6.8586x; 6.8586x over previous
import functools

import jax
import jax.numpy as jnp
from jax import lax
from jax.experimental import pallas as pl
from jax.experimental.pallas import tpu as pltpu
from jax.experimental.pallas import tpu_sc as plsc

NC, NS = 2, 16
WIN = 128
NWT = 248
EPAD = NS * NWT * WIN
EBLK = EPAD // WIN

N_A, N_P, N_T = 10000, 50000, 5000
E = 500000
D = 128
BR = 1024

_f32 = jnp.float32


def _rup(x, m):
    return (x + m - 1) // m * m


NPAD = {"author": _rup(N_A, 2048), "paper": _rup(N_P, 2048), "term": _rup(N_T, 2048)}


@functools.lru_cache(maxsize=None)
def _make_spmm(n_srcpad, n_dstpad, cq, nq):
    qp = nq // NC
    stripe = n_dstpad // NS
    mesh = plsc.VectorSubcoreMesh(core_axis_name="c", subcore_axis_name="s")

    CH = 4
    NCHK = NWT // CH

    def body(*refs):
        tabs = refs[:nq]
        src_hbm, dst_hbm, out = refs[nq], refs[nq + 1], refs[nq + 2]
        (is0, is1, id0, id1, rows0, rows1, zbuf,
         sem0, sem1, sem_is, sem_id, shared) = refs[nq + 3:]
        cid = lax.axis_index("c")
        sid = lax.axis_index("s")
        rbase = sid * NWT

        def zrow(r, c):
            for j in range(cq // 16):
                zbuf[r, pl.ds(j * 16, 16)] = jnp.zeros((16,), _f32)
            return c
        lax.fori_loop(0, 128, zrow, 0)

        rows_b = (rows0, rows1)
        sems_b = (sem0, sem1)

        def process(tab, col_off):
            def zk(k, c):
                pltpu.sync_copy(zbuf, shared.at[pl.ds(sid * stripe + k * 128, 128)])
                return c
            lax.fori_loop(0, stripe // 128, zk, 0)
            plsc.subcore_barrier()

            def ld_idx(chunk, sbuf, dbuf):
                pltpu.async_copy(src_hbm.at[pl.ds(rbase + chunk * CH, CH)],
                                 sbuf, sem_is)
                pltpu.async_copy(dst_hbm.at[pl.ds(rbase + chunk * CH, CH)],
                                 dbuf, sem_id)

            def wait_idx(chunk, sbuf, dbuf):
                pltpu.make_async_copy(src_hbm.at[pl.ds(rbase + chunk * CH, CH)],
                                      sbuf, sem_is).wait()
                pltpu.make_async_copy(dst_hbm.at[pl.ds(rbase + chunk * CH, CH)],
                                      dbuf, sem_id).wait()

            def gstart(ibuf, j, buf, sem):
                pltpu.async_copy(tab.at[ibuf.at[j]], buf, sem)

            def gwait(ibuf, j, buf, sem):
                pltpu.make_async_copy(tab.at[ibuf.at[j]], buf, sem).wait()

            ld_idx(0, is0, id0)
            wait_idx(0, is0, id0)
            ld_idx(1, is1, id1)
            gstart(is0, 0, rows0, sem0)
            gstart(is0, 1, rows1, sem1)

            def step(t, carry):
                for half in range(2):
                    c = 2 * t + half
                    is_c, id_c = (is0, id0) if half == 0 else (is1, id1)
                    is_n, id_n = (is1, id1) if half == 0 else (is0, id0)
                    for j in range(CH):
                        b = j % 2
                        gwait(is_c, j, rows_b[b], sems_b[b])
                        pltpu.sync_copy(rows_b[b], shared.at[id_c.at[j]],
                                        add=True)
                        if j < CH - 2:
                            gstart(is_c, j + 2, rows_b[b], sems_b[b])
                        else:
                            @pl.when(c < NCHK - 1)
                            def _(j=j, b=b, c=c, is_n=is_n, id_n=id_n):
                                if j == CH - 2:
                                    wait_idx(c + 1, is_n, id_n)
                                gstart(is_n, j - (CH - 2), rows_b[b],
                                       sems_b[b])

                    @pl.when(c < NCHK - 2)
                    def _(c=c, is_c=is_c, id_c=id_c):
                        ld_idx(c + 2, is_c, id_c)
                return carry
            lax.fori_loop(0, NCHK // 2, step, 0)
            plsc.subcore_barrier()
            pltpu.sync_copy(
                shared.at[pl.ds(sid * stripe, stripe)],
                out.at[pl.ds(sid * stripe, stripe), pl.ds(col_off, cq)])
            plsc.subcore_barrier()

        for c in range(NC):
            @pl.when(cid == c)
            def _(c=c):
                for qi in range(qp):
                    g = c * qp + qi
                    process(tabs[g], g * cq)

    return pl.kernel(
        body,
        out_type=jax.ShapeDtypeStruct((n_dstpad, D), _f32),
        mesh=mesh,
        compiler_params=pltpu.CompilerParams(use_tc_tiling_on_sc=False),
        scratch_types=[
            pltpu.VMEM((CH, WIN), jnp.int32),
            pltpu.VMEM((CH, WIN), jnp.int32),
            pltpu.VMEM((CH, WIN), jnp.int32),
            pltpu.VMEM((CH, WIN), jnp.int32),
            pltpu.VMEM((WIN, cq), _f32),
            pltpu.VMEM((WIN, cq), _f32),
            pltpu.VMEM((128, cq), _f32),
            pltpu.SemaphoreType.DMA,
            pltpu.SemaphoreType.DMA,
            pltpu.SemaphoreType.DMA,
            pltpu.SemaphoreType.DMA,
            pltpu.VMEM_SHARED((n_dstpad, cq), _f32),
        ],
    )


def _spmm(tables, src2d, dst2d, n_dstpad):
    nq = len(tables)
    cq = D // nq
    fn = _make_spmm(tables[0].shape[0], n_dstpad, cq, nq)
    return fn(*tables, src2d, dst2d)


@functools.lru_cache(maxsize=None)
def _make_counts():
    mesh = plsc.VectorSubcoreMesh(core_axis_name="c", subcore_axis_name="s")
    np_p, np_a, np_t = NPAD["paper"], NPAD["author"], NPAD["term"]

    def body(dap, dpa, dpt, dtp, o_ap, o_pa, o_pt, o_tp,
             idx_d, ones, zc, sem, sh_big, sh_small):
        cid = lax.axis_index("c")
        sid = lax.axis_index("s")
        for j in range(8):
            ones[pl.ds(j * 16, 16)] = jnp.ones((16,), _f32)
            zc[pl.ds(j * 16, 16)] = jnp.zeros((16,), _f32)

        def pc(dst_hbm, sh, npad, out):
            stripe = npad // NS
            pltpu.sync_copy(dst_hbm.at[pl.ds(sid * NWT, NWT)], idx_d)

            def zk(k, c):
                pltpu.sync_copy(zc, sh.at[pl.ds(sid * stripe + k * 128, 128)])
                return c
            lax.fori_loop(0, stripe // 128, zk, 0)
            plsc.subcore_barrier()

            def step(t, c):
                for j in range(8):
                    pltpu.async_copy(ones, sh.at[idx_d.at[8 * t + j]], sem, add=True)
                for j in range(8):
                    pltpu.make_async_copy(ones, sh.at[idx_d.at[8 * t + j]], sem).wait()
                return c
            lax.fori_loop(0, NWT // 8, step, 0)
            plsc.subcore_barrier()
            pltpu.sync_copy(sh.at[pl.ds(sid * stripe, stripe)],
                            out.at[pl.ds(sid * stripe, stripe)])
            plsc.subcore_barrier()

        @pl.when(cid == 0)
        def _():
            pc(dap, sh_big, np_p, o_ap)
            pc(dpa, sh_small, np_a, o_pa)

        @pl.when(cid == 1)
        def _():
            pc(dtp, sh_big, np_p, o_tp)
            pc(dpt, sh_small, np_t, o_pt)

    return pl.kernel(
        body,
        out_type=[jax.ShapeDtypeStruct((np_p,), _f32),
                  jax.ShapeDtypeStruct((np_a,), _f32),
                  jax.ShapeDtypeStruct((np_t,), _f32),
                  jax.ShapeDtypeStruct((np_p,), _f32)],
        mesh=mesh,
        scratch_types=[
            pltpu.VMEM((NWT, WIN), jnp.int32),
            pltpu.VMEM((WIN,), _f32),
            pltpu.VMEM((128,), _f32),
            pltpu.SemaphoreType.DMA,
            pltpu.VMEM_SHARED((np_p,), _f32),
            pltpu.VMEM_SHARED((np_a,), _f32),
        ],
    )


def _dense1_body(s_ref, c_ref, x_ref, wl_ref, wr_ref, b_ref, o_ref):
    inv = 1.0 / jnp.maximum(c_ref[...], 1.0)
    m = s_ref[...] * inv[:, None]
    acc = jnp.dot(m, wl_ref[...], preferred_element_type=_f32)
    acc = acc + jnp.dot(x_ref[...], wr_ref[...], preferred_element_type=_f32)
    o_ref[...] = jnp.maximum(acc + b_ref[...][None, :], 0.0)


def _dense2_body(s1_ref, s2_ref, c1_ref, c2_ref, x_ref,
                 wl1_ref, wl2_ref, wr1_ref, wr2_ref, b_ref, o_ref):
    inv1 = 1.0 / jnp.maximum(c1_ref[...], 1.0)
    inv2 = 1.0 / jnp.maximum(c2_ref[...], 1.0)
    m1 = s1_ref[...] * inv1[:, None]
    m2 = s2_ref[...] * inv2[:, None]
    acc = jnp.dot(m1, wl1_ref[...], preferred_element_type=_f32)
    acc = acc + jnp.dot(m2, wl2_ref[...], preferred_element_type=_f32)
    acc = acc + jnp.dot(x_ref[...], wr1_ref[...] + wr2_ref[...],
                        preferred_element_type=_f32)
    o_ref[...] = jnp.maximum(acc + b_ref[...][None, :], 0.0)


@functools.lru_cache(maxsize=None)
def _make_dense1(npad):
    grid = npad // BR
    mat = pl.BlockSpec((BR, D), lambda i: (i, 0))
    vec = pl.BlockSpec((BR,), lambda i: (i,))
    w = pl.BlockSpec((D, D), lambda i: (0, 0))
    b = pl.BlockSpec((D,), lambda i: (0,))
    return pl.pallas_call(
        _dense1_body,
        grid=(grid,),
        in_specs=[mat, vec, mat, w, w, b],
        out_specs=mat,
        out_shape=jax.ShapeDtypeStruct((npad, D), _f32),
    )


@functools.lru_cache(maxsize=None)
def _make_dense2(npad):
    grid = npad // BR
    mat = pl.BlockSpec((BR, D), lambda i: (i, 0))
    vec = pl.BlockSpec((BR,), lambda i: (i,))
    w = pl.BlockSpec((D, D), lambda i: (0, 0))
    b = pl.BlockSpec((D,), lambda i: (0,))
    return pl.pallas_call(
        _dense2_body,
        grid=(grid,),
        in_specs=[mat, mat, vec, vec, mat, w, w, w, w, b],
        out_specs=mat,
        out_shape=jax.ShapeDtypeStruct((npad, D), _f32),
    )


def _head_body(hm_ref, hg_ref, wm_ref, bm_ref, wg_ref, bg_ref, w2_ref, b2_ref,
               o_ref):
    lg = jnp.dot(hm_ref[...], wm_ref[...], preferred_element_type=_f32)
    lg = lg + bm_ref[...][None, :]
    l1 = jnp.dot(hg_ref[...], wg_ref[...], preferred_element_type=_f32)
    l1 = l1 + bg_ref[...][None, :]
    t = jnp.sum(l1 * w2_ref[...], axis=1, keepdims=True) + b2_ref[0]
    o_ref[...] = lg / t


@functools.lru_cache(maxsize=None)
def _make_head(npad, o):
    grid = npad // BR
    mat = pl.BlockSpec((BR, D), lambda i: (i, 0))
    w = pl.BlockSpec((D, o), lambda i: (0, 0))
    b = pl.BlockSpec((o,), lambda i: (0,))
    w2 = pl.BlockSpec((1, o), lambda i: (0, 0))
    b2 = pl.BlockSpec(memory_space=pltpu.SMEM)
    return pl.pallas_call(
        _head_body,
        grid=(grid,),
        in_specs=[mat, mat, w, b, w, b, w2, b2],
        out_specs=pl.BlockSpec((BR, o), lambda i: (i, 0)),
        out_shape=jax.ShapeDtypeStruct((npad, o), _f32),
    )


def _pad_rows(x, npad):
    return jnp.pad(x, ((0, npad - x.shape[0]), (0, 0)))


def _prep_edges(edge, n_src, n_dst):
    pad = EPAD - E
    ar = jnp.arange(pad, dtype=jnp.int32)
    src = jnp.concatenate([edge[0], (ar * 37) % n_src]).reshape(EBLK, WIN)
    dst = jnp.concatenate([edge[1], n_dst + (ar % 8)]).reshape(EBLK, WIN)
    return src, dst


def _layer_sums(xd, ed):
    xa, xp, xt = xd["author"], xd["paper"], xd["term"]
    a_q = [xa[:, 32 * q:32 * (q + 1)] for q in range(4)]
    t_q = [xt[:, 32 * q:32 * (q + 1)] for q in range(4)]
    p_h = [xp[:, :64], xp[:, 64:]]
    return {
        "ap": _spmm(a_q, *ed["ap"], NPAD["paper"]),
        "tp": _spmm(t_q, *ed["tp"], NPAD["paper"]),
        "pa": _spmm(p_h, *ed["pa"], NPAD["author"]),
        "pt": _spmm(p_h, *ed["pt"], NPAD["term"]),
    }


def _dense_layer(sums, cnt, xd, lp):
    d1a = _make_dense1(NPAD["author"])
    d1t = _make_dense1(NPAD["term"])
    d2p = _make_dense2(NPAD["paper"])
    author = d1a(sums["pa"], cnt["pa"], xd["author"],
                 lp["pa"]["Wl"], lp["pa"]["Wr"], lp["pa"]["bl"])
    term = d1t(sums["pt"], cnt["pt"], xd["term"],
               lp["pt"]["Wl"], lp["pt"]["Wr"], lp["pt"]["bl"])
    paper = d2p(sums["ap"], sums["tp"], cnt["ap"], cnt["tp"], xd["paper"],
                lp["ap"]["Wl"], lp["tp"]["Wl"], lp["ap"]["Wr"], lp["tp"]["Wr"],
                lp["ap"]["bl"] + lp["tp"]["bl"])
    return {"author": author, "paper": paper, "term": term}


def kernel(x_author, x_paper, x_term, edge_ap, edge_pa, edge_pt, edge_tp,
           params):
    xd0 = {"author": _pad_rows(x_author, NPAD["author"]),
           "paper": _pad_rows(x_paper, NPAD["paper"]),
           "term": _pad_rows(x_term, NPAD["term"])}
    ed = {"ap": _prep_edges(edge_ap, N_A, N_P),
          "pa": _prep_edges(edge_pa, N_P, N_A),
          "pt": _prep_edges(edge_pt, N_P, N_T),
          "tp": _prep_edges(edge_tp, N_T, N_P)}

    c_ap, c_pa, c_pt, c_tp = _make_counts()(
        ed["ap"][1], ed["pa"][1], ed["pt"][1], ed["tp"][1])
    cnt = {"ap": c_ap, "pa": c_pa, "pt": c_pt, "tp": c_tp}

    mp, gp, cp = params["model"], params["gts"], params["convs"]

    sums1 = _layer_sums(xd0, ed)
    h1m = _dense_layer(sums1, cnt, xd0, mp["layers"][0])
    h1c = _dense_layer(sums1, cnt, xd0, cp[0])

    sums2m = _layer_sums(h1m, ed)
    h2m = _dense_layer(sums2m, cnt, h1m, mp["layers"][1])

    sums2c = _layer_sums(h1c, ed)
    h2c = _dense_layer(sums2c, cnt, h1c, cp[1])

    sumsg1 = _layer_sums(h2m, ed)
    g1 = _dense_layer(sumsg1, cnt, h2m, gp["layers"][0])

    sumsg2 = _layer_sums(g1, ed)
    g2 = _dense_layer(sumsg2, cnt, g1, gp["layers"][1])

    o = params["lin2_W"].shape[0]
    logits = _make_head(NPAD["author"], o)(
        h2m["author"], g2["author"], mp["lin_W"], mp["lin_b"],
        gp["lin_W"], gp["lin_b"], params["lin2_W"].reshape(1, o),
        params["lin2_b"])

    return (logits[:N_A], h2c["author"][:N_A], h2c["paper"][:N_P],
            h2c["term"][:N_T])


# speedup vs baseline: 7.4217x; 1.0821x over previous
import functools

import jax
import jax.numpy as jnp
from jax import lax
from jax.experimental import pallas as pl
from jax.experimental.pallas import tpu as pltpu
from jax.experimental.pallas import tpu_sc as plsc

NC, NS = 2, 16
WIN = 128
NWT = 248
EPAD = NS * NWT * WIN
EBLK = EPAD // WIN

N_A, N_P, N_T = 10000, 50000, 5000
E = 500000
D = 128
BR = 1024

_f32 = jnp.float32


def _rup(x, m):
    return (x + m - 1) // m * m


NPAD = {"author": _rup(N_A, 2048), "paper": _rup(N_P, 2048), "term": _rup(N_T, 2048)}


@functools.lru_cache(maxsize=None)
def _make_spmm(n_srcpad, n_dstpad, cq, nq):
    qp = nq // NC
    stripe = n_dstpad // NS
    mesh = plsc.VectorSubcoreMesh(core_axis_name="c", subcore_axis_name="s")

    CH = 4
    NCHK = NWT // CH

    def body(*refs):
        tabs = refs[:nq]
        src_hbm, dst_hbm, out = refs[nq], refs[nq + 1], refs[nq + 2]
        (is0, is1, id0, id1, r0, r1, r2, r3, x0, x1, x2, x3, zbuf,
         g0, g1, g2, g3, s0, s1, s2, s3, sem_is, sem_id, shared) = refs[nq + 3:]
        cid = lax.axis_index("c")
        sid = lax.axis_index("s")
        rbase = sid * NWT
        rows = (r0, r1, r2, r3)
        sidx = (x0, x1, x2, x3)
        gsem = (g0, g1, g2, g3)
        ssem = (s0, s1, s2, s3)

        def zrow(r, c):
            for j in range(cq // 16):
                zbuf[r, pl.ds(j * 16, 16)] = jnp.zeros((16,), _f32)
            return c
        lax.fori_loop(0, 128, zrow, 0)

        def process(tab, col_off):
            def zk(k, c):
                pltpu.sync_copy(zbuf, shared.at[pl.ds(sid * stripe + k * 128, 128)])
                return c
            lax.fori_loop(0, stripe // 128, zk, 0)
            plsc.subcore_barrier()

            def ld_idx(chunk, sbuf, dbuf):
                pltpu.async_copy(src_hbm.at[pl.ds(rbase + chunk * CH, CH)],
                                 sbuf, sem_is)
                pltpu.async_copy(dst_hbm.at[pl.ds(rbase + chunk * CH, CH)],
                                 dbuf, sem_id)

            def wait_idx(chunk, sbuf, dbuf):
                pltpu.make_async_copy(src_hbm.at[pl.ds(rbase + chunk * CH, CH)],
                                      sbuf, sem_is).wait()
                pltpu.make_async_copy(dst_hbm.at[pl.ds(rbase + chunk * CH, CH)],
                                      dbuf, sem_id).wait()

            def gstart(ibuf, j, slot):
                pltpu.async_copy(tab.at[ibuf.at[j]], rows[slot], gsem[slot])

            def gwait(ibuf, j, slot):
                pltpu.make_async_copy(tab.at[ibuf.at[j]], rows[slot],
                                      gsem[slot]).wait()

            def sstart(ibuf, j, slot):
                for k in range(WIN // 16):
                    sidx[slot][pl.ds(k * 16, 16)] = ibuf[j, pl.ds(k * 16, 16)]
                pltpu.async_copy(rows[slot], shared.at[sidx[slot]],
                                 ssem[slot], add=True)

            def swait(slot):
                pltpu.make_async_copy(rows[slot], shared.at[sidx[slot]],
                                      ssem[slot]).wait()

            ld_idx(0, is0, id0)
            wait_idx(0, is0, id0)
            ld_idx(1, is1, id1)
            gstart(is0, 0, 0)
            gstart(is0, 1, 1)

            def step(t, carry):
                for half in range(2):
                    c = 2 * t + half
                    is_c, id_c = (is0, id0) if half == 0 else (is1, id1)
                    is_n, id_n = (is1, id1) if half == 0 else (is0, id0)
                    for j in range(CH):
                        gwait(is_c, j, j)
                        sstart(id_c, j, j)
                        if j < 2:
                            @pl.when(c > 0)
                            def _(j=j):
                                swait(j + 2)
                            gstart(is_c, j + 2, j + 2)
                        else:
                            @pl.when(c < NCHK - 1)
                            def _(j=j, c=c, is_n=is_n, id_n=id_n):
                                if j == 2:
                                    wait_idx(c + 1, is_n, id_n)
                                swait(j - 2)
                                gstart(is_n, j - 2, j - 2)

                    @pl.when(c < NCHK - 2)
                    def _(c=c, is_c=is_c, id_c=id_c):
                        ld_idx(c + 2, is_c, id_c)
                return carry
            lax.fori_loop(0, NCHK // 2, step, 0)
            for slot in range(4):
                swait(slot)
            plsc.subcore_barrier()
            pltpu.sync_copy(
                shared.at[pl.ds(sid * stripe, stripe)],
                out.at[pl.ds(sid * stripe, stripe), pl.ds(col_off, cq)])
            plsc.subcore_barrier()

        for c in range(NC):
            @pl.when(cid == c)
            def _(c=c):
                for qi in range(qp):
                    g = c * qp + qi
                    process(tabs[g], g * cq)

    return pl.kernel(
        body,
        out_type=jax.ShapeDtypeStruct((n_dstpad, D), _f32),
        mesh=mesh,
        compiler_params=pltpu.CompilerParams(use_tc_tiling_on_sc=False),
        scratch_types=[
            pltpu.VMEM((CH, WIN), jnp.int32),
            pltpu.VMEM((CH, WIN), jnp.int32),
            pltpu.VMEM((CH, WIN), jnp.int32),
            pltpu.VMEM((CH, WIN), jnp.int32),
            pltpu.VMEM((WIN, cq), _f32),
            pltpu.VMEM((WIN, cq), _f32),
            pltpu.VMEM((WIN, cq), _f32),
            pltpu.VMEM((WIN, cq), _f32),
            pltpu.VMEM((WIN,), jnp.int32),
            pltpu.VMEM((WIN,), jnp.int32),
            pltpu.VMEM((WIN,), jnp.int32),
            pltpu.VMEM((WIN,), jnp.int32),
            pltpu.VMEM((128, cq), _f32),
            pltpu.SemaphoreType.DMA,
            pltpu.SemaphoreType.DMA,
            pltpu.SemaphoreType.DMA,
            pltpu.SemaphoreType.DMA,
            pltpu.SemaphoreType.DMA,
            pltpu.SemaphoreType.DMA,
            pltpu.SemaphoreType.DMA,
            pltpu.SemaphoreType.DMA,
            pltpu.SemaphoreType.DMA,
            pltpu.SemaphoreType.DMA,
            pltpu.VMEM_SHARED((n_dstpad, cq), _f32),
        ],
    )


def _spmm(tables, src2d, dst2d, n_dstpad):
    nq = len(tables)
    cq = D // nq
    fn = _make_spmm(tables[0].shape[0], n_dstpad, cq, nq)
    return fn(*tables, src2d, dst2d)


@functools.lru_cache(maxsize=None)
def _make_counts():
    mesh = plsc.VectorSubcoreMesh(core_axis_name="c", subcore_axis_name="s")
    np_p, np_a, np_t = NPAD["paper"], NPAD["author"], NPAD["term"]

    def body(dap, dpa, dpt, dtp, o_ap, o_pa, o_pt, o_tp,
             idx_d, ones, zc, sem, sh_big, sh_small):
        cid = lax.axis_index("c")
        sid = lax.axis_index("s")
        for j in range(8):
            ones[pl.ds(j * 16, 16)] = jnp.ones((16,), _f32)
            zc[pl.ds(j * 16, 16)] = jnp.zeros((16,), _f32)

        def pc(dst_hbm, sh, npad, out):
            stripe = npad // NS
            pltpu.sync_copy(dst_hbm.at[pl.ds(sid * NWT, NWT)], idx_d)

            def zk(k, c):
                pltpu.sync_copy(zc, sh.at[pl.ds(sid * stripe + k * 128, 128)])
                return c
            lax.fori_loop(0, stripe // 128, zk, 0)
            plsc.subcore_barrier()

            def step(t, c):
                for j in range(8):
                    pltpu.async_copy(ones, sh.at[idx_d.at[8 * t + j]], sem, add=True)
                for j in range(8):
                    pltpu.make_async_copy(ones, sh.at[idx_d.at[8 * t + j]], sem).wait()
                return c
            lax.fori_loop(0, NWT // 8, step, 0)
            plsc.subcore_barrier()
            pltpu.sync_copy(sh.at[pl.ds(sid * stripe, stripe)],
                            out.at[pl.ds(sid * stripe, stripe)])
            plsc.subcore_barrier()

        @pl.when(cid == 0)
        def _():
            pc(dap, sh_big, np_p, o_ap)
            pc(dpa, sh_small, np_a, o_pa)

        @pl.when(cid == 1)
        def _():
            pc(dtp, sh_big, np_p, o_tp)
            pc(dpt, sh_small, np_t, o_pt)

    return pl.kernel(
        body,
        out_type=[jax.ShapeDtypeStruct((np_p,), _f32),
                  jax.ShapeDtypeStruct((np_a,), _f32),
                  jax.ShapeDtypeStruct((np_t,), _f32),
                  jax.ShapeDtypeStruct((np_p,), _f32)],
        mesh=mesh,
        scratch_types=[
            pltpu.VMEM((NWT, WIN), jnp.int32),
            pltpu.VMEM((WIN,), _f32),
            pltpu.VMEM((128,), _f32),
            pltpu.SemaphoreType.DMA,
            pltpu.VMEM_SHARED((np_p,), _f32),
            pltpu.VMEM_SHARED((np_a,), _f32),
        ],
    )


def _dense1_body(s_ref, c_ref, x_ref, wl_ref, wr_ref, b_ref, o_ref):
    inv = 1.0 / jnp.maximum(c_ref[...], 1.0)
    m = s_ref[...] * inv[:, None]
    acc = jnp.dot(m, wl_ref[...], preferred_element_type=_f32)
    acc = acc + jnp.dot(x_ref[...], wr_ref[...], preferred_element_type=_f32)
    o_ref[...] = jnp.maximum(acc + b_ref[...][None, :], 0.0)


def _dense2_body(s1_ref, s2_ref, c1_ref, c2_ref, x_ref,
                 wl1_ref, wl2_ref, wr1_ref, wr2_ref, b_ref, o_ref):
    inv1 = 1.0 / jnp.maximum(c1_ref[...], 1.0)
    inv2 = 1.0 / jnp.maximum(c2_ref[...], 1.0)
    m1 = s1_ref[...] * inv1[:, None]
    m2 = s2_ref[...] * inv2[:, None]
    acc = jnp.dot(m1, wl1_ref[...], preferred_element_type=_f32)
    acc = acc + jnp.dot(m2, wl2_ref[...], preferred_element_type=_f32)
    acc = acc + jnp.dot(x_ref[...], wr1_ref[...] + wr2_ref[...],
                        preferred_element_type=_f32)
    o_ref[...] = jnp.maximum(acc + b_ref[...][None, :], 0.0)


@functools.lru_cache(maxsize=None)
def _make_dense1(npad):
    grid = npad // BR
    mat = pl.BlockSpec((BR, D), lambda i: (i, 0))
    vec = pl.BlockSpec((BR,), lambda i: (i,))
    w = pl.BlockSpec((D, D), lambda i: (0, 0))
    b = pl.BlockSpec((D,), lambda i: (0,))
    return pl.pallas_call(
        _dense1_body,
        grid=(grid,),
        in_specs=[mat, vec, mat, w, w, b],
        out_specs=mat,
        out_shape=jax.ShapeDtypeStruct((npad, D), _f32),
    )


@functools.lru_cache(maxsize=None)
def _make_dense2(npad):
    grid = npad // BR
    mat = pl.BlockSpec((BR, D), lambda i: (i, 0))
    vec = pl.BlockSpec((BR,), lambda i: (i,))
    w = pl.BlockSpec((D, D), lambda i: (0, 0))
    b = pl.BlockSpec((D,), lambda i: (0,))
    return pl.pallas_call(
        _dense2_body,
        grid=(grid,),
        in_specs=[mat, mat, vec, vec, mat, w, w, w, w, b],
        out_specs=mat,
        out_shape=jax.ShapeDtypeStruct((npad, D), _f32),
    )


def _head_body(hm_ref, hg_ref, wm_ref, bm_ref, wg_ref, bg_ref, w2_ref, b2_ref,
               o_ref):
    lg = jnp.dot(hm_ref[...], wm_ref[...], preferred_element_type=_f32)
    lg = lg + bm_ref[...][None, :]
    l1 = jnp.dot(hg_ref[...], wg_ref[...], preferred_element_type=_f32)
    l1 = l1 + bg_ref[...][None, :]
    t = jnp.sum(l1 * w2_ref[...], axis=1, keepdims=True) + b2_ref[0]
    o_ref[...] = lg / t


@functools.lru_cache(maxsize=None)
def _make_head(npad, o):
    grid = npad // BR
    mat = pl.BlockSpec((BR, D), lambda i: (i, 0))
    w = pl.BlockSpec((D, o), lambda i: (0, 0))
    b = pl.BlockSpec((o,), lambda i: (0,))
    w2 = pl.BlockSpec((1, o), lambda i: (0, 0))
    b2 = pl.BlockSpec(memory_space=pltpu.SMEM)
    return pl.pallas_call(
        _head_body,
        grid=(grid,),
        in_specs=[mat, mat, w, b, w, b, w2, b2],
        out_specs=pl.BlockSpec((BR, o), lambda i: (i, 0)),
        out_shape=jax.ShapeDtypeStruct((npad, o), _f32),
    )


def _pad_rows(x, npad):
    return jnp.pad(x, ((0, npad - x.shape[0]), (0, 0)))


def _prep_edges(edge, n_src, n_dst):
    pad = EPAD - E
    ar = jnp.arange(pad, dtype=jnp.int32)
    src = jnp.concatenate([edge[0], (ar * 37) % n_src]).reshape(EBLK, WIN)
    dst = jnp.concatenate([edge[1], n_dst + (ar % 8)]).reshape(EBLK, WIN)
    return src, dst


def _layer_sums(xd, ed):
    xa, xp, xt = xd["author"], xd["paper"], xd["term"]
    a_q = [xa[:, 32 * q:32 * (q + 1)] for q in range(4)]
    t_q = [xt[:, 32 * q:32 * (q + 1)] for q in range(4)]
    p_h = [xp[:, :64], xp[:, 64:]]
    return {
        "ap": _spmm(a_q, *ed["ap"], NPAD["paper"]),
        "tp": _spmm(t_q, *ed["tp"], NPAD["paper"]),
        "pa": _spmm(p_h, *ed["pa"], NPAD["author"]),
        "pt": _spmm(p_h, *ed["pt"], NPAD["term"]),
    }


def _dense_layer(sums, cnt, xd, lp):
    d1a = _make_dense1(NPAD["author"])
    d1t = _make_dense1(NPAD["term"])
    d2p = _make_dense2(NPAD["paper"])
    author = d1a(sums["pa"], cnt["pa"], xd["author"],
                 lp["pa"]["Wl"], lp["pa"]["Wr"], lp["pa"]["bl"])
    term = d1t(sums["pt"], cnt["pt"], xd["term"],
               lp["pt"]["Wl"], lp["pt"]["Wr"], lp["pt"]["bl"])
    paper = d2p(sums["ap"], sums["tp"], cnt["ap"], cnt["tp"], xd["paper"],
                lp["ap"]["Wl"], lp["tp"]["Wl"], lp["ap"]["Wr"], lp["tp"]["Wr"],
                lp["ap"]["bl"] + lp["tp"]["bl"])
    return {"author": author, "paper": paper, "term": term}


def kernel(x_author, x_paper, x_term, edge_ap, edge_pa, edge_pt, edge_tp,
           params):
    xd0 = {"author": _pad_rows(x_author, NPAD["author"]),
           "paper": _pad_rows(x_paper, NPAD["paper"]),
           "term": _pad_rows(x_term, NPAD["term"])}
    ed = {"ap": _prep_edges(edge_ap, N_A, N_P),
          "pa": _prep_edges(edge_pa, N_P, N_A),
          "pt": _prep_edges(edge_pt, N_P, N_T),
          "tp": _prep_edges(edge_tp, N_T, N_P)}

    c_ap, c_pa, c_pt, c_tp = _make_counts()(
        ed["ap"][1], ed["pa"][1], ed["pt"][1], ed["tp"][1])
    cnt = {"ap": c_ap, "pa": c_pa, "pt": c_pt, "tp": c_tp}

    mp, gp, cp = params["model"], params["gts"], params["convs"]

    sums1 = _layer_sums(xd0, ed)
    h1m = _dense_layer(sums1, cnt, xd0, mp["layers"][0])
    h1c = _dense_layer(sums1, cnt, xd0, cp[0])

    sums2m = _layer_sums(h1m, ed)
    h2m = _dense_layer(sums2m, cnt, h1m, mp["layers"][1])

    sums2c = _layer_sums(h1c, ed)
    h2c = _dense_layer(sums2c, cnt, h1c, cp[1])

    sumsg1 = _layer_sums(h2m, ed)
    g1 = _dense_layer(sumsg1, cnt, h2m, gp["layers"][0])

    sumsg2 = _layer_sums(g1, ed)
    g2 = _dense_layer(sumsg2, cnt, g1, gp["layers"][1])

    o = params["lin2_W"].shape[0]
    logits = _make_head(NPAD["author"], o)(
        h2m["author"], g2["author"], mp["lin_W"], mp["lin_b"],
        gp["lin_W"], gp["lin_b"], params["lin2_W"].reshape(1, o),
        params["lin2_b"])

    return (logits[:N_A], h2c["author"][:N_A], h2c["paper"][:N_P],
            h2c["term"][:N_T])


# speedup vs baseline: 7.4238x; 1.0003x over previous
import functools

import jax
import jax.numpy as jnp
from jax import lax
from jax.experimental import pallas as pl
from jax.experimental.pallas import tpu as pltpu
from jax.experimental.pallas import tpu_sc as plsc

NC, NS = 2, 16
WIN = 128
NWT = 248
EPAD = NS * NWT * WIN
EBLK = EPAD // WIN

N_A, N_P, N_T = 10000, 50000, 5000
E = 500000
D = 128
BR = 1024

_f32 = jnp.float32


def _rup(x, m):
    return (x + m - 1) // m * m


NPAD = {"author": _rup(N_A, 2048), "paper": _rup(N_P, 2048), "term": _rup(N_T, 2048)}


@functools.lru_cache(maxsize=None)
def _make_spmm(n_srcpad, n_dstpad, cq, nq):
    qp = nq // NC
    stripe = n_dstpad // NS
    mesh = plsc.VectorSubcoreMesh(core_axis_name="c", subcore_axis_name="s")

    CH = 4
    NCHK = NWT // CH

    def body(*refs):
        tabs = refs[:nq]
        src_hbm, dst_hbm, out = refs[nq], refs[nq + 1], refs[nq + 2]
        (is0, is1, id0, id1, r0, r1, r2, r3, x0, x1, x2, x3, zbuf,
         g0, g1, g2, g3, s0, s1, s2, s3, sem_is, sem_id, shared) = refs[nq + 3:]
        cid = lax.axis_index("c")
        sid = lax.axis_index("s")
        rbase = sid * NWT
        rows = (r0, r1, r2, r3)
        sidx = (x0, x1, x2, x3)
        gsem = (g0, g1, g2, g3)
        ssem = (s0, s1, s2, s3)

        def zrow(r, c):
            for j in range(cq // 16):
                zbuf[r, pl.ds(j * 16, 16)] = jnp.zeros((16,), _f32)
            return c
        lax.fori_loop(0, 128, zrow, 0)

        def process(tab, col_off):
            def zk(k, c):
                pltpu.sync_copy(zbuf, shared.at[pl.ds(sid * stripe + k * 128, 128)])
                return c
            lax.fori_loop(0, stripe // 128, zk, 0)
            plsc.subcore_barrier()

            def ld_idx(chunk, sbuf, dbuf):
                pltpu.async_copy(src_hbm.at[pl.ds(rbase + chunk * CH, CH)],
                                 sbuf, sem_is)
                pltpu.async_copy(dst_hbm.at[pl.ds(rbase + chunk * CH, CH)],
                                 dbuf, sem_id)

            def wait_idx(chunk, sbuf, dbuf):
                pltpu.make_async_copy(src_hbm.at[pl.ds(rbase + chunk * CH, CH)],
                                      sbuf, sem_is).wait()
                pltpu.make_async_copy(dst_hbm.at[pl.ds(rbase + chunk * CH, CH)],
                                      dbuf, sem_id).wait()

            def gstart(ibuf, j, slot):
                pltpu.async_copy(tab.at[ibuf.at[j]], rows[slot], gsem[slot])

            def gwait(ibuf, j, slot):
                pltpu.make_async_copy(tab.at[ibuf.at[j]], rows[slot],
                                      gsem[slot]).wait()

            def sstart(ibuf, j, slot):
                for k in range(WIN // 16):
                    sidx[slot][pl.ds(k * 16, 16)] = ibuf[j, pl.ds(k * 16, 16)]
                pltpu.async_copy(rows[slot], shared.at[sidx[slot]],
                                 ssem[slot], add=True)

            def swait(slot):
                pltpu.make_async_copy(rows[slot], shared.at[sidx[slot]],
                                      ssem[slot]).wait()

            ld_idx(0, is0, id0)
            wait_idx(0, is0, id0)
            ld_idx(1, is1, id1)
            gstart(is0, 0, 0)
            gstart(is0, 1, 1)

            def step(t, carry):
                for half in range(2):
                    c = 2 * t + half
                    is_c, id_c = (is0, id0) if half == 0 else (is1, id1)
                    is_n, id_n = (is1, id1) if half == 0 else (is0, id0)
                    for j in range(CH):
                        gwait(is_c, j, j)
                        sstart(id_c, j, j)
                        if j < 2:
                            @pl.when(c > 0)
                            def _(j=j):
                                swait(j + 2)
                            gstart(is_c, j + 2, j + 2)
                        else:
                            @pl.when(c < NCHK - 1)
                            def _(j=j, c=c, is_n=is_n, id_n=id_n):
                                if j == 2:
                                    wait_idx(c + 1, is_n, id_n)
                                swait(j - 2)
                                gstart(is_n, j - 2, j - 2)

                    @pl.when(c < NCHK - 2)
                    def _(c=c, is_c=is_c, id_c=id_c):
                        ld_idx(c + 2, is_c, id_c)
                return carry
            lax.fori_loop(0, NCHK // 2, step, 0)
            for slot in range(4):
                swait(slot)
            plsc.subcore_barrier()
            pltpu.sync_copy(
                shared.at[pl.ds(sid * stripe, stripe)],
                out.at[pl.ds(sid * stripe, stripe), pl.ds(col_off, cq)])
            plsc.subcore_barrier()

        for c in range(NC):
            @pl.when(cid == c)
            def _(c=c):
                for qi in range(qp):
                    g = c * qp + qi
                    process(tabs[g], g * cq)

    return pl.kernel(
        body,
        out_type=jax.ShapeDtypeStruct((n_dstpad, D), _f32),
        mesh=mesh,
        compiler_params=pltpu.CompilerParams(use_tc_tiling_on_sc=False),
        scratch_types=[
            pltpu.VMEM((CH, WIN), jnp.int32),
            pltpu.VMEM((CH, WIN), jnp.int32),
            pltpu.VMEM((CH, WIN), jnp.int32),
            pltpu.VMEM((CH, WIN), jnp.int32),
            pltpu.VMEM((WIN, cq), _f32),
            pltpu.VMEM((WIN, cq), _f32),
            pltpu.VMEM((WIN, cq), _f32),
            pltpu.VMEM((WIN, cq), _f32),
            pltpu.VMEM((WIN,), jnp.int32),
            pltpu.VMEM((WIN,), jnp.int32),
            pltpu.VMEM((WIN,), jnp.int32),
            pltpu.VMEM((WIN,), jnp.int32),
            pltpu.VMEM((128, cq), _f32),
            pltpu.SemaphoreType.DMA,
            pltpu.SemaphoreType.DMA,
            pltpu.SemaphoreType.DMA,
            pltpu.SemaphoreType.DMA,
            pltpu.SemaphoreType.DMA,
            pltpu.SemaphoreType.DMA,
            pltpu.SemaphoreType.DMA,
            pltpu.SemaphoreType.DMA,
            pltpu.SemaphoreType.DMA,
            pltpu.SemaphoreType.DMA,
            pltpu.VMEM_SHARED((n_dstpad, cq), _f32),
        ],
    )


def _spmm(tables, src2d, dst2d, n_dstpad):
    nq = len(tables)
    cq = D // nq
    fn = _make_spmm(tables[0].shape[0], n_dstpad, cq, nq)
    return fn(*tables, src2d, dst2d)


@functools.lru_cache(maxsize=None)
def _make_counts():
    mesh = plsc.VectorSubcoreMesh(core_axis_name="c", subcore_axis_name="s")
    np_p, np_a, np_t = NPAD["paper"], NPAD["author"], NPAD["term"]

    def body(dap, dpa, dpt, dtp, o_ap, o_pa, o_pt, o_tp,
             idx_d, ones, zc, sem, sh_big, sh_small):
        cid = lax.axis_index("c")
        sid = lax.axis_index("s")
        for j in range(8):
            ones[pl.ds(j * 16, 16)] = jnp.ones((16,), _f32)
            zc[pl.ds(j * 16, 16)] = jnp.zeros((16,), _f32)

        def pc(dst_hbm, sh, npad, out):
            stripe = npad // NS
            pltpu.sync_copy(dst_hbm.at[pl.ds(sid * NWT, NWT)], idx_d)

            def zk(k, c):
                pltpu.sync_copy(zc, sh.at[pl.ds(sid * stripe + k * 128, 128)])
                return c
            lax.fori_loop(0, stripe // 128, zk, 0)
            plsc.subcore_barrier()

            def step(t, c):
                for j in range(8):
                    pltpu.async_copy(ones, sh.at[idx_d.at[8 * t + j]], sem, add=True)
                for j in range(8):
                    pltpu.make_async_copy(ones, sh.at[idx_d.at[8 * t + j]], sem).wait()
                return c
            lax.fori_loop(0, NWT // 8, step, 0)
            plsc.subcore_barrier()
            pltpu.sync_copy(sh.at[pl.ds(sid * stripe, stripe)],
                            out.at[pl.ds(sid * stripe, stripe)])
            plsc.subcore_barrier()

        @pl.when(cid == 0)
        def _():
            pc(dap, sh_big, np_p, o_ap)
            pc(dpa, sh_small, np_a, o_pa)

        @pl.when(cid == 1)
        def _():
            pc(dtp, sh_big, np_p, o_tp)
            pc(dpt, sh_small, np_t, o_pt)

    return pl.kernel(
        body,
        out_type=[jax.ShapeDtypeStruct((np_p,), _f32),
                  jax.ShapeDtypeStruct((np_a,), _f32),
                  jax.ShapeDtypeStruct((np_t,), _f32),
                  jax.ShapeDtypeStruct((np_p,), _f32)],
        mesh=mesh,
        scratch_types=[
            pltpu.VMEM((NWT, WIN), jnp.int32),
            pltpu.VMEM((WIN,), _f32),
            pltpu.VMEM((128,), _f32),
            pltpu.SemaphoreType.DMA,
            pltpu.VMEM_SHARED((np_p,), _f32),
            pltpu.VMEM_SHARED((np_a,), _f32),
        ],
    )


def _dense1_body(s_ref, c_ref, x_ref, wl_ref, wr_ref, b_ref, o_ref):
    inv = 1.0 / jnp.maximum(c_ref[...], 1.0)
    m = s_ref[...] * inv[:, None]
    acc = jnp.dot(m, wl_ref[...], preferred_element_type=_f32)
    acc = acc + jnp.dot(x_ref[...], wr_ref[...], preferred_element_type=_f32)
    o_ref[...] = jnp.maximum(acc + b_ref[...][None, :], 0.0)


def _dense2_body(s1_ref, s2_ref, c1_ref, c2_ref, x_ref,
                 wl1_ref, wl2_ref, wr1_ref, wr2_ref, b_ref, o_ref):
    inv1 = 1.0 / jnp.maximum(c1_ref[...], 1.0)
    inv2 = 1.0 / jnp.maximum(c2_ref[...], 1.0)
    m1 = s1_ref[...] * inv1[:, None]
    m2 = s2_ref[...] * inv2[:, None]
    acc = jnp.dot(m1, wl1_ref[...], preferred_element_type=_f32)
    acc = acc + jnp.dot(m2, wl2_ref[...], preferred_element_type=_f32)
    acc = acc + jnp.dot(x_ref[...], wr1_ref[...] + wr2_ref[...],
                        preferred_element_type=_f32)
    o_ref[...] = jnp.maximum(acc + b_ref[...][None, :], 0.0)


@functools.lru_cache(maxsize=None)
def _make_dense1(npad):
    grid = npad // BR
    mat = pl.BlockSpec((BR, D), lambda i: (i, 0))
    vec = pl.BlockSpec((BR,), lambda i: (i,))
    w = pl.BlockSpec((D, D), lambda i: (0, 0))
    b = pl.BlockSpec((D,), lambda i: (0,))
    return pl.pallas_call(
        _dense1_body,
        grid=(grid,),
        in_specs=[mat, vec, mat, w, w, b],
        out_specs=mat,
        out_shape=jax.ShapeDtypeStruct((npad, D), _f32),
    )


@functools.lru_cache(maxsize=None)
def _make_dense2(npad):
    grid = npad // BR
    mat = pl.BlockSpec((BR, D), lambda i: (i, 0))
    vec = pl.BlockSpec((BR,), lambda i: (i,))
    w = pl.BlockSpec((D, D), lambda i: (0, 0))
    b = pl.BlockSpec((D,), lambda i: (0,))
    return pl.pallas_call(
        _dense2_body,
        grid=(grid,),
        in_specs=[mat, mat, vec, vec, mat, w, w, w, w, b],
        out_specs=mat,
        out_shape=jax.ShapeDtypeStruct((npad, D), _f32),
    )


def _head_body(hm_ref, hg_ref, wm_ref, bm_ref, wg_ref, bg_ref, w2_ref, b2_ref,
               o_ref):
    lg = jnp.dot(hm_ref[...], wm_ref[...], preferred_element_type=_f32)
    lg = lg + bm_ref[...][None, :]
    l1 = jnp.dot(hg_ref[...], wg_ref[...], preferred_element_type=_f32)
    l1 = l1 + bg_ref[...][None, :]
    t = jnp.sum(l1 * w2_ref[...], axis=1, keepdims=True) + b2_ref[0]
    o_ref[...] = lg / t


@functools.lru_cache(maxsize=None)
def _make_head(npad, o):
    grid = npad // BR
    mat = pl.BlockSpec((BR, D), lambda i: (i, 0))
    w = pl.BlockSpec((D, o), lambda i: (0, 0))
    b = pl.BlockSpec((o,), lambda i: (0,))
    w2 = pl.BlockSpec((1, o), lambda i: (0, 0))
    b2 = pl.BlockSpec(memory_space=pltpu.SMEM)
    return pl.pallas_call(
        _head_body,
        grid=(grid,),
        in_specs=[mat, mat, w, b, w, b, w2, b2],
        out_specs=pl.BlockSpec((BR, o), lambda i: (i, 0)),
        out_shape=jax.ShapeDtypeStruct((npad, o), _f32),
    )


def _pad_rows(x, npad):
    return jnp.pad(x, ((0, npad - x.shape[0]), (0, 0)))


def _prep_edges(edge, n_src, n_dst):
    pad = EPAD - E
    ar = jnp.arange(pad, dtype=jnp.int32)
    src = jnp.concatenate([edge[0], (ar * 37) % n_src]).reshape(EBLK, WIN)
    dst = jnp.concatenate([edge[1], n_dst + (ar % 8)]).reshape(EBLK, WIN)
    return src, dst


def _layer_sums(xd, ed):
    xa, xp, xt = xd["author"], xd["paper"], xd["term"]
    a_q = [xa[:, 32 * q:32 * (q + 1)] for q in range(4)]
    t_q = [xt[:, 32 * q:32 * (q + 1)] for q in range(4)]
    p_h = [xp[:, :64], xp[:, 64:]]
    return {
        "ap": _spmm(a_q, *ed["ap"], NPAD["paper"]),
        "tp": _spmm(t_q, *ed["tp"], NPAD["paper"]),
        "pa": _spmm(p_h, *ed["pa"], NPAD["author"]),
        "pt": _spmm(p_h, *ed["pt"], NPAD["term"]),
    }


def _dense_layer(sums, cnt, xd, lp):
    d1a = _make_dense1(NPAD["author"])
    d1t = _make_dense1(NPAD["term"])
    d2p = _make_dense2(NPAD["paper"])
    author = d1a(sums["pa"], cnt["pa"], xd["author"],
                 lp["pa"]["Wl"], lp["pa"]["Wr"], lp["pa"]["bl"])
    term = d1t(sums["pt"], cnt["pt"], xd["term"],
               lp["pt"]["Wl"], lp["pt"]["Wr"], lp["pt"]["bl"])
    paper = d2p(sums["ap"], sums["tp"], cnt["ap"], cnt["tp"], xd["paper"],
                lp["ap"]["Wl"], lp["tp"]["Wl"], lp["ap"]["Wr"], lp["tp"]["Wr"],
                lp["ap"]["bl"] + lp["tp"]["bl"])
    return {"author": author, "paper": paper, "term": term}


def kernel(x_author, x_paper, x_term, edge_ap, edge_pa, edge_pt, edge_tp,
           params):
    xd0 = {"author": _pad_rows(x_author, NPAD["author"]),
           "paper": _pad_rows(x_paper, NPAD["paper"]),
           "term": _pad_rows(x_term, NPAD["term"])}
    ed = {"ap": _prep_edges(edge_ap, N_A, N_P),
          "pa": _prep_edges(edge_pa, N_P, N_A),
          "pt": _prep_edges(edge_pt, N_P, N_T),
          "tp": _prep_edges(edge_tp, N_T, N_P)}

    c_ap, c_pa, c_pt, c_tp = _make_counts()(
        ed["ap"][1], ed["pa"][1], ed["pt"][1], ed["tp"][1])
    cnt = {"ap": c_ap, "pa": c_pa, "pt": c_pt, "tp": c_tp}

    mp, gp, cp = params["model"], params["gts"], params["convs"]

    sums1 = _layer_sums(xd0, ed)
    h1m = _dense_layer(sums1, cnt, xd0, mp["layers"][0])
    h1c = _dense_layer(sums1, cnt, xd0, cp[0])

    sums2m = _layer_sums(h1m, ed)
    h2m = _dense_layer(sums2m, cnt, h1m, mp["layers"][1])

    sums2c = _layer_sums(h1c, ed)
    h2c = _dense_layer(sums2c, cnt, h1c, cp[1])

    sumsg1 = _layer_sums(h2m, ed)
    g1 = _dense_layer(sumsg1, cnt, h2m, gp["layers"][0])

    xp = g1["paper"]
    s_pa_g2 = _spmm([xp[:, :64], xp[:, 64:]], *ed["pa"], NPAD["author"])
    lp2 = gp["layers"][1]["pa"]
    g2_author = _make_dense1(NPAD["author"])(
        s_pa_g2, cnt["pa"], g1["author"], lp2["Wl"], lp2["Wr"], lp2["bl"])
    g2 = {"author": g2_author}

    o = params["lin2_W"].shape[0]
    logits = _make_head(NPAD["author"], o)(
        h2m["author"], g2["author"], mp["lin_W"], mp["lin_b"],
        gp["lin_W"], gp["lin_b"], params["lin2_W"].reshape(1, o),
        params["lin2_b"])

    return (logits[:N_A], h2c["author"][:N_A], h2c["paper"][:N_P],
            h2c["term"][:N_T])
